# baseline (speedup 1.0000x reference)
import jax
import jax.numpy as jnp
from jax import lax
from jax.experimental import pallas as pl

B, E_IN, C, D, K, V = 256, 768, 8, 64, 1024, 256


def _tc_body(batch_ref, rpT_ref, wT_ref, b_ref, cbT_ref, e2_ref, val_ref,
             out_ref):
    c = pl.program_id(0)
    x = batch_ref[...]
    rpT = rpT_ref[0]
    tq = lax.dot_general(x, rpT, (((1,), (1,)), ((), ())),
                         preferred_element_type=jnp.float32)
    y = lax.dot_general(tq, wT_ref[...], (((1,), (1,)), ((), ())),
                        preferred_element_type=jnp.float32) \
        + b_ref[pl.ds(c, 1), :]
    cbT = cbT_ref[0]
    xe = jnp.dot(y, cbT, preferred_element_type=jnp.float32)
    x2 = jnp.sum(y * y, axis=1, keepdims=True)
    dist = -(x2 - 2.0 * xe + e2_ref[0])
    m = jnp.max(dist, axis=1, keepdims=True)
    kidx = lax.broadcasted_iota(jnp.int32, (B, K), 1)
    idx = jnp.min(jnp.where(dist == m, kidx, K), axis=1, keepdims=True)
    ones = jnp.full((V,), 1.0 / V, dtype=jnp.float32)
    vmean = lax.dot_general(val_ref[0], ones, (((1,), (0,)), ((), ())),
                            preferred_element_type=jnp.float32)
    sel = jnp.where(kidx == idx, vmean[None, :], 0.0)
    col = jnp.sum(sel, axis=1, keepdims=True)
    lane = lax.broadcasted_iota(jnp.int32, (B, C), 1)
    out_ref[...] = jnp.where(lane == c, col, out_ref[...])


@jax.jit
def kernel(batch, values, rand_proj, W_in, b_in, codebook):
    out = pl.pallas_call(
        _tc_body,
        grid=(C,),
        in_specs=[
            pl.BlockSpec((B, E_IN), lambda c: (0, 0)),
            pl.BlockSpec((1, D, E_IN), lambda c: (c, 0, 0)),
            pl.BlockSpec((D, D), lambda c: (c, 0)),
            pl.BlockSpec((C, D), lambda c: (0, 0)),
            pl.BlockSpec((1, D, K), lambda c: (c, 0, 0)),
            pl.BlockSpec((1, 1, K), lambda c: (c, 0, 0)),
            pl.BlockSpec((1, K, V), lambda c: (c, 0, 0)),
        ],
        out_specs=pl.BlockSpec((B, C), lambda c: (0, 0)),
        out_shape=jax.ShapeDtypeStruct((B, C), jnp.float32),
    )(batch, rand_proj.transpose(0, 2, 1), W_in, b_in.reshape(C, D),
      codebook.transpose(0, 2, 1),
      jnp.sum(codebook * codebook, axis=-1).reshape(C, 1, K), values)
    return out
